# baseline (speedup 1.0000x reference)
import functools

import jax
import jax.numpy as jnp
from jax import lax
from jax.experimental import pallas as pl
from jax.experimental.pallas import tpu as pltpu
from jax.experimental.pallas import tpu_sc as plsc

HID = 128
EPS = 1e-12

_SC_INFO = plsc.get_sparse_core_info()
_NC = _SC_INFO.num_cores
_NS = _SC_INFO.num_subcores
_NW = _NC * _NS

_CHUNK = 128


def _gather_body(tok0, seq, ids_hbm, table_hbm, out_hbm,
                 idx0, rows0, gsem0, wsem):
    wid = lax.axis_index("s") * _NC + lax.axis_index("c")
    flat = tok0 + wid * _CHUNK
    row = flat // seq
    col = flat % seq
    pltpu.sync_copy(ids_hbm.at[row, pl.ds(col, _CHUNK)], idx0)
    g0 = pltpu.async_copy(table_hbm.at[idx0], rows0, gsem0)
    g0.wait()
    w0 = pltpu.async_copy(rows0, out_hbm.at[pl.ds(wid * _CHUNK, _CHUNK)],
                          wsem)
    w0.wait()


def _sc_gather_half(ids, word_emb, tok0):
    seq = ids.shape[1]
    n_half = _NW * _CHUNK
    mesh = plsc.VectorSubcoreMesh(core_axis_name="c", subcore_axis_name="s")
    k = functools.partial(
        pl.kernel,
        mesh=mesh,
        out_type=jax.ShapeDtypeStruct((n_half, HID), jnp.float32),
        scratch_types=[
            pltpu.VMEM((_CHUNK,), jnp.int32),
            pltpu.VMEM((_CHUNK, HID), jnp.float32),
            pltpu.SemaphoreType.DMA,
            pltpu.SemaphoreType.DMA,
        ],
    )(functools.partial(_gather_body, tok0, seq))
    return k(ids, word_emb)


def _tc_body(x_ref, pos_ref, type_ref, gamma_ref, beta_ref, w_ref, b_ref,
             out_ref):
    x = x_ref[...] + pos_ref[...] + type_ref[0:1, :]
    mean = jnp.mean(x, axis=-1, keepdims=True)
    xc = x - mean
    var = jnp.mean(xc * xc, axis=-1, keepdims=True)
    normed = xc * lax.rsqrt(var + EPS)
    y = normed * gamma_ref[...] + beta_ref[...]
    out_ref[...] = lax.dot_general(
        y, w_ref[...], (((1,), (1,)), ((), ())),
        preferred_element_type=jnp.float32) + b_ref[...]


def _tc_half(gathered, pos_emb, type_emb, gamma, beta, W, bias, seq):
    n_rows = gathered.shape[0]
    n_blk = n_rows // seq
    return pl.pallas_call(
        _tc_body,
        grid=(n_blk,),
        in_specs=[
            pl.BlockSpec((seq, HID), lambda i: (i, 0)),
            pl.BlockSpec((seq, HID), lambda i: (0, 0)),
            pl.BlockSpec((2, HID), lambda i: (0, 0)),
            pl.BlockSpec((1, HID), lambda i: (0, 0)),
            pl.BlockSpec((1, HID), lambda i: (0, 0)),
            pl.BlockSpec((HID, HID), lambda i: (0, 0)),
            pl.BlockSpec((1, HID), lambda i: (0, 0)),
        ],
        out_specs=pl.BlockSpec((seq, HID), lambda i: (i, 0)),
        out_shape=jax.ShapeDtypeStruct((n_rows, HID), jnp.float32),
    )(gathered, pos_emb, type_emb, gamma, beta, W, bias)


def kernel(input_ids, word_emb, pos_emb, type_emb, ln_gamma, ln_beta, W, b):
    batch, seq = input_ids.shape
    n_tokens = batch * seq
    n_half = _NW * _CHUNK
    assert n_tokens == 2 * n_half

    gamma = ln_gamma.reshape(1, HID)
    beta = ln_beta.reshape(1, HID)
    bias = b.reshape(1, HID)

    g0 = _sc_gather_half(input_ids, word_emb, 0)
    g1 = _sc_gather_half(input_ids, word_emb, n_half)
    out0 = _tc_half(g0, pos_emb, type_emb, gamma, beta, W, bias, seq)
    out1 = _tc_half(g1, pos_emb, type_emb, gamma, beta, W, bias, seq)

    out = jnp.concatenate([out0, out1], axis=0)
    return out.reshape(batch, seq, HID)


# speedup vs baseline: 1.0075x; 1.0075x over previous
import functools

import jax
import jax.numpy as jnp
from jax import lax
from jax.experimental import pallas as pl
from jax.experimental.pallas import tpu as pltpu
from jax.experimental.pallas import tpu_sc as plsc

HID = 128
EPS = 1e-12

_SC_INFO = plsc.get_sparse_core_info()
_NC = _SC_INFO.num_cores
_NS = _SC_INFO.num_subcores
_NW = _NC * _NS

_CHUNK = 128


def _gather_body(chunks_per_row, ids_hbm, table_hbm, out_hbm,
                 idx0, idx1, rows0, rows1, gsem0, gsem1, wsem):
    wid = lax.axis_index("s") * _NC + lax.axis_index("c")
    row = wid // chunks_per_row
    col = (wid % chunks_per_row) * (2 * _CHUNK)
    base = wid * (2 * _CHUNK)
    pltpu.sync_copy(ids_hbm.at[row, pl.ds(col, _CHUNK)], idx0)
    g0 = pltpu.async_copy(table_hbm.at[idx0], rows0, gsem0)
    pltpu.sync_copy(ids_hbm.at[row, pl.ds(col + _CHUNK, _CHUNK)], idx1)
    g1 = pltpu.async_copy(table_hbm.at[idx1], rows1, gsem1)
    g0.wait()
    w0 = pltpu.async_copy(rows0, out_hbm.at[pl.ds(base, _CHUNK)], wsem)
    g1.wait()
    w1 = pltpu.async_copy(rows1, out_hbm.at[pl.ds(base + _CHUNK, _CHUNK)],
                          wsem)
    w0.wait()
    w1.wait()


def _sc_gather(ids, word_emb):
    batch, seq = ids.shape
    n_tokens = batch * seq
    assert n_tokens == _NW * 2 * _CHUNK
    chunks_per_row = seq // (2 * _CHUNK)
    mesh = plsc.VectorSubcoreMesh(core_axis_name="c", subcore_axis_name="s")
    k = functools.partial(
        pl.kernel,
        mesh=mesh,
        out_type=jax.ShapeDtypeStruct((n_tokens, HID), jnp.float32),
        scratch_types=[
            pltpu.VMEM((_CHUNK,), jnp.int32),
            pltpu.VMEM((_CHUNK,), jnp.int32),
            pltpu.VMEM((_CHUNK, HID), jnp.float32),
            pltpu.VMEM((_CHUNK, HID), jnp.float32),
            pltpu.SemaphoreType.DMA,
            pltpu.SemaphoreType.DMA,
            pltpu.SemaphoreType.DMA,
        ],
    )(functools.partial(_gather_body, chunks_per_row))
    return k(ids, word_emb)


def _tc_body(blk, seq, x_ref, pos_ref, type_ref, gamma_ref, beta_ref, w_ref,
             b_ref, out_ref):
    per_seq = seq // blk
    s = lax.rem(pl.program_id(0), per_seq) * blk
    x = x_ref[...] + pos_ref[pl.ds(s, blk), :] + type_ref[0:1, :]
    mean = jnp.mean(x, axis=-1, keepdims=True)
    xc = x - mean
    var = jnp.mean(xc * xc, axis=-1, keepdims=True)
    normed = xc * lax.rsqrt(var + EPS)
    y = normed * gamma_ref[...] + beta_ref[...]
    out_ref[...] = lax.dot_general(
        y, w_ref[...], (((1,), (1,)), ((), ())),
        preferred_element_type=jnp.float32) + b_ref[...]


def kernel(input_ids, word_emb, pos_emb, type_emb, ln_gamma, ln_beta, W, b):
    batch, seq = input_ids.shape
    gathered = _sc_gather(input_ids, word_emb)

    gamma = ln_gamma.reshape(1, HID)
    beta = ln_beta.reshape(1, HID)
    bias = b.reshape(1, HID)

    blk = 512
    n_blk = (batch * seq) // blk
    out = pl.pallas_call(
        functools.partial(_tc_body, blk, seq),
        grid=(n_blk,),
        in_specs=[
            pl.BlockSpec((blk, HID), lambda i: (i, 0)),
            pl.BlockSpec((seq, HID), lambda i: (0, 0)),
            pl.BlockSpec((2, HID), lambda i: (0, 0)),
            pl.BlockSpec((1, HID), lambda i: (0, 0)),
            pl.BlockSpec((1, HID), lambda i: (0, 0)),
            pl.BlockSpec((HID, HID), lambda i: (0, 0)),
            pl.BlockSpec((1, HID), lambda i: (0, 0)),
        ],
        out_specs=pl.BlockSpec((blk, HID), lambda i: (i, 0)),
        out_shape=jax.ShapeDtypeStruct((batch * seq, HID), jnp.float32),
    )(gathered, pos_emb, type_emb, gamma, beta, W, bias)

    return out.reshape(batch, seq, HID)


# speedup vs baseline: 1.2311x; 1.2220x over previous
import functools

import jax
import jax.numpy as jnp
from jax import lax
from jax.experimental import pallas as pl
from jax.experimental.pallas import tpu as pltpu
from jax.experimental.pallas import tpu_sc as plsc

HID = 128
EPS = 1e-12

_SC_INFO = plsc.get_sparse_core_info()
_NC = _SC_INFO.num_cores
_NS = _SC_INFO.num_subcores
_NW = _NC * _NS

_CHUNK = 128


def _gather_body(chunks_per_row, ids_hbm, table_hbm, out_hbm,
                 idx0, idx1, rows0, rows1, gsem0, gsem1, wsem):
    wid = lax.axis_index("s") * _NC + lax.axis_index("c")
    row = wid // chunks_per_row
    col = (wid % chunks_per_row) * (2 * _CHUNK)
    base = wid * (2 * _CHUNK)
    pltpu.sync_copy(ids_hbm.at[row, pl.ds(col, _CHUNK)], idx0)
    g0 = pltpu.async_copy(table_hbm.at[idx0], rows0, gsem0)
    pltpu.sync_copy(ids_hbm.at[row, pl.ds(col + _CHUNK, _CHUNK)], idx1)
    g1 = pltpu.async_copy(table_hbm.at[idx1], rows1, gsem1)
    g0.wait()
    w0 = pltpu.async_copy(rows0, out_hbm.at[pl.ds(base, _CHUNK)], wsem)
    g1.wait()
    w1 = pltpu.async_copy(rows1, out_hbm.at[pl.ds(base + _CHUNK, _CHUNK)],
                          wsem)
    w0.wait()
    w1.wait()


def _sc_gather(ids, word_emb):
    batch, seq = ids.shape
    n_tokens = batch * seq
    assert n_tokens == _NW * 2 * _CHUNK
    chunks_per_row = seq // (2 * _CHUNK)
    mesh = plsc.VectorSubcoreMesh(core_axis_name="c", subcore_axis_name="s")
    k = functools.partial(
        pl.kernel,
        mesh=mesh,
        out_type=jax.ShapeDtypeStruct((n_tokens, HID), jnp.float32),
        scratch_types=[
            pltpu.VMEM((_CHUNK,), jnp.int32),
            pltpu.VMEM((_CHUNK,), jnp.int32),
            pltpu.VMEM((_CHUNK, HID), jnp.float32),
            pltpu.VMEM((_CHUNK, HID), jnp.float32),
            pltpu.SemaphoreType.DMA,
            pltpu.SemaphoreType.DMA,
            pltpu.SemaphoreType.DMA,
        ],
    )(functools.partial(_gather_body, chunks_per_row))
    return k(ids, word_emb)


def _tc_body(blk, seq, x_ref, pos_ref, type_ref, gamma_ref, beta_ref, w_ref,
             b_ref, out_ref):
    per_seq = seq // blk
    s = lax.rem(pl.program_id(0), per_seq) * blk
    x = x_ref[...] + pos_ref[pl.ds(s, blk), :] + type_ref[0:1, :]
    mean = jnp.mean(x, axis=-1, keepdims=True)
    xc = x - mean
    var = jnp.mean(xc * xc, axis=-1, keepdims=True)
    normed = xc * lax.rsqrt(var + EPS)
    y = normed * gamma_ref[...] + beta_ref[...]
    out_ref[...] = lax.dot_general(
        y, w_ref[...], (((1,), (1,)), ((), ())),
        preferred_element_type=jnp.float32) + b_ref[...]


def kernel(input_ids, word_emb, pos_emb, type_emb, ln_gamma, ln_beta, W, b):
    batch, seq = input_ids.shape
    gathered = _sc_gather(input_ids, word_emb)

    gamma = ln_gamma.reshape(1, HID)
    beta = ln_beta.reshape(1, HID)
    bias = b.reshape(1, HID)

    blk = 2048
    n_blk = (batch * seq) // blk
    out = pl.pallas_call(
        functools.partial(_tc_body, blk, seq),
        grid=(n_blk,),
        in_specs=[
            pl.BlockSpec((blk, HID), lambda i: (i, 0)),
            pl.BlockSpec((seq, HID), lambda i: (0, 0)),
            pl.BlockSpec((2, HID), lambda i: (0, 0)),
            pl.BlockSpec((1, HID), lambda i: (0, 0)),
            pl.BlockSpec((1, HID), lambda i: (0, 0)),
            pl.BlockSpec((HID, HID), lambda i: (0, 0)),
            pl.BlockSpec((1, HID), lambda i: (0, 0)),
        ],
        out_specs=pl.BlockSpec((blk, HID), lambda i: (i, 0)),
        out_shape=jax.ShapeDtypeStruct((batch * seq, HID), jnp.float32),
    )(gathered, pos_emb, type_emb, gamma, beta, W, bias)

    return out.reshape(batch, seq, HID)
